# Initial kernel scaffold; baseline (speedup 1.0000x reference)
import jax
import jax.numpy as jnp
from jax.experimental import pallas as pl


def kernel(input_ids, old_word_embeddings, new_word_embeddings, position_embeddings, token_type_embeddings, ln_weight, ln_bias):
    raise NotImplementedError("write your pallas kernel here")


# speedup vs baseline: 1.5407x; 1.5407x over previous
import jax
import jax.numpy as jnp
from jax import lax
from jax.experimental import pallas as pl
from jax.experimental.pallas import tpu as pltpu
from jax.experimental.pallas import tpu_sc as plsc

OLD_VOCAB = 25005
NEW_VOCAB = 224257
HIDDEN = 768
PAD_IDX = 1
LN_EPS = 1e-12
B, S = 1024, 50
NC, NS = 2, 16
NW = NC * NS
ROWS_W = B // NW
TOK_W = ROWS_W * S
K = 16
NCHUNK = TOK_W // K
NPOS = 56
NV = HIDDEN // 16


def _take16(x, idx):
    dnums = lax.GatherDimensionNumbers(
        offset_dims=(), collapsed_slice_dims=(0,), start_index_map=(0,))
    return lax.gather(x, idx[:, None], dnums, (1,),
                      mode=lax.GatherScatterMode.PROMISE_IN_BOUNDS)


def _body(idsT, old_tbl, new_tbl, pos_tbl, typ_tbl, lnw, lnb, out,
          idsT_v, posid_v, oldidx_v, newidx_v, outidx2, ptt_v, tt_v, w_v, b_v,
          obuf, nbuf, sem_g0, sem_g1, sem_o):
    cid = lax.axis_index("c")
    sid = lax.axis_index("s")
    wid = sid * NC + cid
    lanes = lax.iota(jnp.int32, 16)

    pltpu.sync_copy(idsT.at[wid], idsT_v)
    pltpu.sync_copy(pos_tbl.at[pl.ds(0, NPOS)], ptt_v)
    pltpu.sync_copy(typ_tbl, tt_v)
    pltpu.sync_copy(lnw, w_v)
    pltpu.sync_copy(lnb, b_v)

    def fold(r, carry):
        for j in range(NV):
            sl = pl.ds(j * 16, 16)
            ptt_v[r, sl] = ptt_v[r, sl] + tt_v[sl]
        return carry
    lax.fori_loop(0, NPOS, fold, 0)

    out_base = wid * TOK_W
    for g in range(ROWS_W // 16):
        orow = out_base + (lanes + g * 16) * S
        acc = jnp.zeros((16,), jnp.int32)
        for s in range(S):
            c = g * S + s
            v = idsT_v[pl.ds(s * ROWS_W + g * 16, 16)]
            m = jnp.minimum(jnp.abs(v - PAD_IDX), 1)
            acc = acc + m
            posid = acc * m + PAD_IDX
            d = v - OLD_VOCAB
            so = lax.shift_right_logical(d, 31)
            sl = pl.ds(c * K, 16)
            posid_v[sl] = posid
            oldidx_v[sl] = 1 + so * (v - 1)
            newidx_v[sl] = 1 + (1 - so) * (d - 1)
            outidx2[c, :] = orow + s

    def chunk(c, carry):
        g1 = pltpu.async_copy(old_tbl.at[oldidx_v.at[pl.ds(c * K, K)]],
                              obuf, sem_g0)
        g2 = pltpu.async_copy(new_tbl.at[newidx_v.at[pl.ds(c * K, K)]],
                              nbuf, sem_g1)
        pv = posid_v[pl.ds(c * K, 16)]
        ps = [pv[t] for t in range(K)]
        g1.wait()
        g2.wait()

        def addpos(j, jcarry):
            sl = pl.ds(j * 16, 16)
            for t in range(K):
                obuf[t, sl] = obuf[t, sl] + nbuf[t, sl] + ptt_v[ps[t], sl]
            return jcarry
        lax.fori_loop(0, NV, addpos, 0)

        def token(t, tcarry):
            acc1 = jnp.zeros((16,), jnp.float32)
            acc2 = jnp.zeros((16,), jnp.float32)
            for j in range(NV):
                sl = pl.ds(j * 16, 16)
                v = obuf[t, sl]
                acc1 = acc1 + v
                acc2 = acc2 + v * v
            for k in (8, 4, 2, 1):
                acc1 = acc1 + _take16(acc1, lanes ^ k)
                acc2 = acc2 + _take16(acc2, lanes ^ k)
            mean = acc1 * (1.0 / HIDDEN)
            var = acc2 * (1.0 / HIDDEN) - mean * mean + LN_EPS
            i = lax.bitcast_convert_type(var, jnp.int32)
            y = lax.bitcast_convert_type(jnp.int32(0x5F3759DF) - (i >> 1),
                                         jnp.float32)
            for _ in range(3):
                y = y * (1.5 - 0.5 * var * y * y)
            for j in range(NV):
                sl = pl.ds(j * 16, 16)
                o = (obuf[t, sl] - mean) * y
                obuf[t, sl] = o * w_v[sl] + b_v[sl]
            return tcarry
        lax.fori_loop(0, K, token, 0)

        pltpu.async_copy(obuf, out.at[outidx2.at[c]], sem_o).wait()
        return carry

    lax.fori_loop(0, NCHUNK, chunk, 0)


def kernel(input_ids, old_word_embeddings, new_word_embeddings,
           position_embeddings, token_type_embeddings, ln_weight, ln_bias):
    ids = input_ids.astype(jnp.int32)
    idsT = ids.reshape(NW, ROWS_W, S).transpose(0, 2, 1).reshape(NW, TOK_W)
    mesh = plsc.VectorSubcoreMesh(core_axis_name="c", subcore_axis_name="s")
    scratch = [
        pltpu.VMEM((TOK_W,), jnp.int32),
        pltpu.VMEM((TOK_W,), jnp.int32),
        pltpu.VMEM((TOK_W,), jnp.int32),
        pltpu.VMEM((TOK_W,), jnp.int32),
        pltpu.VMEM((NCHUNK, K), jnp.int32),
        pltpu.VMEM((NPOS, HIDDEN), jnp.float32),
        pltpu.VMEM((2 * HIDDEN,), jnp.float32),
        pltpu.VMEM((HIDDEN,), jnp.float32),
        pltpu.VMEM((HIDDEN,), jnp.float32),
        pltpu.VMEM((K, HIDDEN), jnp.float32),
        pltpu.VMEM((K, HIDDEN), jnp.float32),
        pltpu.SemaphoreType.DMA,
        pltpu.SemaphoreType.DMA,
        pltpu.SemaphoreType.DMA,
    ]
    f = pl.kernel(
        _body,
        out_type=jax.ShapeDtypeStruct((B * S, HIDDEN), jnp.float32),
        mesh=mesh,
        scratch_types=scratch,
    )
    out = f(idsT, old_word_embeddings, new_word_embeddings,
            position_embeddings, token_type_embeddings.reshape(2 * HIDDEN),
            ln_weight, ln_bias)
    return out.reshape(B, S, HIDDEN)
